# baseline (speedup 1.0000x reference)
import functools

import jax
import jax.numpy as jnp
from jax import lax
from jax.experimental import pallas as pl
from jax.experimental.pallas import tpu as pltpu
from jax.experimental.pallas import tpu_sc as plsc

N = 10000
D = 128
E = 320000
NC = 2
NS = 16
NW = NC * NS
CHUNK = 128
NSLOT = 2
ROWS = E // CHUNK
GR = 8
RPAD = ((ROWS + NW * GR - 1) // (NW * GR)) * (NW * GR)
RPF = RPAD + 2 * GR
EPAD = RPAD * CHUNK
RPW = RPAD // NW
NBLK = RPW // GR
NSP = 10112
ZR = NSP // NS

_mesh = plsc.VectorSubcoreMesh(
    core_axis_name="core", subcore_axis_name="subcore",
    num_cores=NC, num_subcores=NS)


def _zero_rows(ref, nrows, width):
    zero = jnp.zeros((1, 16), jnp.float32)

    @pl.loop(0, nrows)
    def _(r):
        @pl.loop(0, width // 16)
        def _(c):
            ref[pl.ds(r, 1), pl.ds(c * 16, 16)] = zero


def _zero_shared_slice(src, dst, base, total):
    off = 0
    while off < total:
        n = min(CHUNK, total - off)
        pltpu.sync_copy(src.at[pl.ds(0, n)], dst.at[pl.ds(base + off, n)])
        off += n


def _sc_agg_body(h_hbm, es_hbm, ed_hbm, part_hbm, ibs0, ibd0, ibs1, ibd1,
                 rows, agg_sp, sem_i0, sem_i1, sem_g0, sem_g1, sem_s0,
                 sem_s1):
    cidx = lax.axis_index("core")
    sidx = lax.axis_index("subcore")
    wid = cidx * NS + sidx
    wbase = wid * RPW

    _zero_rows(rows, CHUNK, D)
    base = sidx * ZR
    _zero_shared_slice(rows, agg_sp, base, ZR)
    plsc.subcore_barrier()

    sem_g = (sem_g0, sem_g1)
    sem_s = (sem_s0, sem_s1)

    def run_block(row0, ibs, ibd):
        hg = {}
        hs = {}
        hg[0] = pltpu.async_copy(
            h_hbm.at[ibs.at[0]], rows.at[pl.ds(0, CHUNK)], sem_g[0])
        for k in range(GR):
            slot = k % NSLOT
            if k + 1 < GR:
                nslot = (k + 1) % NSLOT
                if k - 1 >= 0:
                    hs[k - 1].wait()
                hg[k + 1] = pltpu.async_copy(
                    h_hbm.at[ibs.at[k + 1]],
                    rows.at[pl.ds(nslot * CHUNK, CHUNK)], sem_g[nslot])
            hg[k].wait()
            hs[k] = pltpu.async_copy(
                rows.at[pl.ds(slot * CHUNK, CHUNK)],
                agg_sp.at[ibd.at[k]], sem_s[slot], add=True)
        hs[GR - 2].wait()
        hs[GR - 1].wait()

    pltpu.sync_copy(es_hbm.at[pl.ds(wbase, GR)], ibs0)
    pltpu.sync_copy(ed_hbm.at[pl.ds(wbase, GR)], ibd0)
    pltpu.async_copy(es_hbm.at[pl.ds(wbase + GR, GR)], ibs1, sem_i1)
    pltpu.async_copy(ed_hbm.at[pl.ds(wbase + GR, GR)], ibd1, sem_i1)

    @pl.loop(0, NBLK, step=2)
    def _(b):
        row0 = wbase + b * GR

        @pl.when(b > 0)
        def _():
            pltpu.make_async_copy(es_hbm.at[pl.ds(row0, GR)], ibs0,
                                  sem_i0).wait()
            pltpu.make_async_copy(ed_hbm.at[pl.ds(row0, GR)], ibd0,
                                  sem_i0).wait()

        run_block(row0, ibs0, ibd0)
        pltpu.async_copy(es_hbm.at[pl.ds(row0 + 2 * GR, GR)], ibs0, sem_i0)
        pltpu.async_copy(ed_hbm.at[pl.ds(row0 + 2 * GR, GR)], ibd0, sem_i0)

        row1 = row0 + GR
        pltpu.make_async_copy(es_hbm.at[pl.ds(row1, GR)], ibs1, sem_i1).wait()
        pltpu.make_async_copy(ed_hbm.at[pl.ds(row1, GR)], ibd1, sem_i1).wait()
        run_block(row1, ibs1, ibd1)
        pltpu.async_copy(es_hbm.at[pl.ds(row1 + 2 * GR, GR)], ibs1, sem_i1)
        pltpu.async_copy(ed_hbm.at[pl.ds(row1 + 2 * GR, GR)], ibd1, sem_i1)

    pltpu.make_async_copy(es_hbm.at[pl.ds(wbase, GR)], ibs0, sem_i0).wait()
    pltpu.make_async_copy(ed_hbm.at[pl.ds(wbase, GR)], ibd0, sem_i0).wait()
    pltpu.make_async_copy(es_hbm.at[pl.ds(wbase, GR)], ibs1, sem_i1).wait()
    pltpu.make_async_copy(ed_hbm.at[pl.ds(wbase, GR)], ibd1, sem_i1).wait()

    plsc.subcore_barrier()
    pltpu.sync_copy(agg_sp.at[pl.ds(base, ZR)],
                    part_hbm.at[cidx].at[pl.ds(base, ZR)])


_sc_agg = pl.kernel(
    _sc_agg_body,
    out_type=jax.ShapeDtypeStruct((NC, NSP, D), jnp.float32),
    mesh=_mesh,
    scratch_types=[
        pltpu.VMEM((GR, CHUNK), jnp.int32),
        pltpu.VMEM((GR, CHUNK), jnp.int32),
        pltpu.VMEM((GR, CHUNK), jnp.int32),
        pltpu.VMEM((GR, CHUNK), jnp.int32),
        pltpu.VMEM((NSLOT * CHUNK, D), jnp.float32),
        pltpu.VMEM_SHARED((NSP, D), jnp.float32),
        pltpu.SemaphoreType.DMA,
        pltpu.SemaphoreType.DMA,
        pltpu.SemaphoreType.DMA,
        pltpu.SemaphoreType.DMA,
        pltpu.SemaphoreType.DMA,
        pltpu.SemaphoreType.DMA,
    ],
)


def _sc_deg_body(ed_hbm, degp_hbm, ibd0, ibd1, ones_b, deg_sp, sem_i0,
                 sem_i1, sem_s):
    cidx = lax.axis_index("core")
    sidx = lax.axis_index("subcore")
    wid = cidx * NS + sidx
    wbase = wid * RPW

    _zero_rows(ones_b, CHUNK, D)
    base = sidx * ZR
    _zero_shared_slice(ones_b, deg_sp, base, ZR)
    one = jnp.ones((1, 16), jnp.float32)

    @pl.loop(0, CHUNK)
    def _(r):
        @pl.loop(0, D // 16)
        def _(c):
            ones_b[pl.ds(r, 1), pl.ds(c * 16, 16)] = one

    plsc.subcore_barrier()

    def run_block(ibd):
        hs = []
        for k in range(GR):
            hs.append(pltpu.async_copy(ones_b, deg_sp.at[ibd.at[k]], sem_s,
                                       add=True))
        for h in hs:
            h.wait()

    pltpu.sync_copy(ed_hbm.at[pl.ds(wbase, GR)], ibd0)
    pltpu.async_copy(ed_hbm.at[pl.ds(wbase + GR, GR)], ibd1, sem_i1)

    @pl.loop(0, NBLK, step=2)
    def _(b):
        row0 = wbase + b * GR

        @pl.when(b > 0)
        def _():
            pltpu.make_async_copy(ed_hbm.at[pl.ds(row0, GR)], ibd0,
                                  sem_i0).wait()

        run_block(ibd0)
        pltpu.async_copy(ed_hbm.at[pl.ds(row0 + 2 * GR, GR)], ibd0, sem_i0)

        row1 = row0 + GR
        pltpu.make_async_copy(ed_hbm.at[pl.ds(row1, GR)], ibd1, sem_i1).wait()
        run_block(ibd1)
        pltpu.async_copy(ed_hbm.at[pl.ds(row1 + 2 * GR, GR)], ibd1, sem_i1)

    pltpu.make_async_copy(ed_hbm.at[pl.ds(wbase, GR)], ibd0, sem_i0).wait()
    pltpu.make_async_copy(ed_hbm.at[pl.ds(wbase, GR)], ibd1, sem_i1).wait()

    plsc.subcore_barrier()
    pltpu.sync_copy(deg_sp.at[pl.ds(base, ZR)],
                    degp_hbm.at[cidx].at[pl.ds(base, ZR)])


_sc_deg = pl.kernel(
    _sc_deg_body,
    out_type=jax.ShapeDtypeStruct((NC, NSP, D), jnp.float32),
    mesh=_mesh,
    scratch_types=[
        pltpu.VMEM((GR, CHUNK), jnp.int32),
        pltpu.VMEM((GR, CHUNK), jnp.int32),
        pltpu.VMEM((CHUNK, D), jnp.float32),
        pltpu.VMEM_SHARED((NSP, D), jnp.float32),
        pltpu.SemaphoreType.DMA,
        pltpu.SemaphoreType.DMA,
        pltpu.SemaphoreType.DMA,
    ],
)


RB = 1000
_PREC = lax.Precision.HIGHEST


def _tc_pre_body(h_ref, w_ref, b_ref, o_ref):
    o_ref[...] = lax.dot_general(
        h_ref[...], w_ref[...], (((1,), (0,)), ((), ())),
        precision=_PREC, preferred_element_type=jnp.float32) + b_ref[...]


def _tc_pre(h, w, b):
    return pl.pallas_call(
        _tc_pre_body,
        grid=(N // RB,),
        in_specs=[
            pl.BlockSpec((RB, D), lambda i: (i, 0)),
            pl.BlockSpec((D, D), lambda i: (0, 0)),
            pl.BlockSpec((1, D), lambda i: (0, 0)),
        ],
        out_specs=pl.BlockSpec((RB, D), lambda i: (i, 0)),
        out_shape=jax.ShapeDtypeStruct((N, D), jnp.float32),
    )(h, w[:D], b.reshape(1, D))


def _tc_post_body(relu, pre_ref, p_ref, d_ref, w_ref, o_ref):
    agg = p_ref[0] + p_ref[1]
    dsum = d_ref[0] + d_ref[1]
    deg = jnp.maximum(dsum[:, 0:1], 1.0)
    mean = agg / deg
    y = pre_ref[...] + lax.dot_general(
        mean, w_ref[...], (((1,), (0,)), ((), ())),
        precision=_PREC, preferred_element_type=jnp.float32)
    if relu:
        y = jnp.maximum(y, 0.0)
    o_ref[...] = y


def _tc_post(pre, part, degp, w, relu):
    return pl.pallas_call(
        functools.partial(_tc_post_body, relu),
        grid=(N // RB,),
        in_specs=[
            pl.BlockSpec((RB, D), lambda i: (i, 0)),
            pl.BlockSpec((NC, RB, D), lambda i: (0, i, 0)),
            pl.BlockSpec((NC, RB, D), lambda i: (0, i, 0)),
            pl.BlockSpec((D, D), lambda i: (0, 0)),
        ],
        out_specs=pl.BlockSpec((RB, D), lambda i: (i, 0)),
        out_shape=jax.ShapeDtypeStruct((N, D), jnp.float32),
    )(pre, part, degp, w[D:])


def kernel(x, edge_index, W0, b0, W1, b1):
    ei = edge_index.astype(jnp.int32)
    pr = RPF - ROWS
    r2 = (lax.broadcasted_iota(jnp.int32, (pr, CHUNK), 0) * CHUNK
          + lax.broadcasted_iota(jnp.int32, (pr, CHUNK), 1))
    pad_s = r2 % N
    pad_d = N + (r2 % (NSP - N))
    es = jnp.concatenate([ei[0].reshape(ROWS, CHUNK), pad_s], axis=0)
    ed = jnp.concatenate([ei[1].reshape(ROWS, CHUNK), pad_d], axis=0)

    degp = _sc_deg(ed)
    part0 = _sc_agg(x, es, ed)
    pre0 = _tc_pre(x, W0, b0)
    h1 = _tc_post(pre0, part0, degp, W0, relu=True)
    part1 = _sc_agg(h1, es, ed)
    pre1 = _tc_pre(h1, W1, b1)
    h2 = _tc_post(pre1, part1, degp, W1, relu=False)
    return h2


# speedup vs baseline: 1.1710x; 1.1710x over previous
import dataclasses
import functools

import jax
import jax.numpy as jnp
from jax import lax
from jax.experimental import pallas as pl
from jax.experimental.pallas import tpu as pltpu
from jax.experimental.pallas import tpu_sc as plsc

N = 10000
D = 128
E = 320000
NC = 2
NS = 16
NW = NC * NS
CHUNK = 128
NSLOT = 2
ROWS = E // CHUNK
GR = 8
RPAD = ((ROWS + NW * GR - 1) // (NW * GR)) * (NW * GR)
RPF = RPAD + 2 * GR
RPW = RPAD // NW
NBLK = RPW // GR
NSP = 10112
ZR = NSP // NS
HR = 80
DR = HR // NS

_mesh = plsc.VectorSubcoreMesh(
    core_axis_name="core", subcore_axis_name="subcore",
    num_cores=NC, num_subcores=NS)

_cp = pltpu.CompilerParams()
if "needs_layout_passes" in pltpu.CompilerParams.__dataclass_fields__:
    _cp = dataclasses.replace(_cp, needs_layout_passes=False)


def _zero_rows(ref, nrows, width):
    zero = jnp.zeros((16,), jnp.float32)

    @pl.loop(0, nrows)
    def _(r):
        for c in range(width // 16):
            ref[r, pl.ds(c * 16, 16)] = zero


def _zero_shared_slice(src, dst, base, total):
    off = 0
    while off < total:
        n = min(CHUNK, total - off)
        pltpu.sync_copy(src.at[pl.ds(0, n)], dst.at[pl.ds(base + off, n)])
        off += n


def _sc_agg_body(with_deg, h_hbm, es_hbm, ed_hbm, ident_hbm, *rest):
    if with_deg:
        (part_hbm, degd_hbm, ibs0, ibd0, ibs1, ibd1, rows, ident, hist,
         agg_sp, deg_sh, sem_i0, sem_i1, sem_g0, sem_g1, sem_s0,
         sem_s1) = rest
    else:
        (part_hbm, ibs0, ibd0, ibs1, ibd1, rows, agg_sp, sem_i0, sem_i1,
         sem_g0, sem_g1, sem_s0, sem_s1) = rest
    cidx = lax.axis_index("core")
    sidx = lax.axis_index("subcore")
    wid = cidx * NS + sidx
    wbase = wid * RPW

    _zero_rows(rows, CHUNK, D)
    base = sidx * ZR
    _zero_shared_slice(rows, agg_sp, base, ZR)
    if with_deg:
        _zero_rows(hist, HR, D)

        @pl.when(sidx == 0)
        def _():
            pltpu.sync_copy(rows.at[pl.ds(0, HR)], deg_sh)

        pltpu.sync_copy(ident_hbm, ident)
    plsc.subcore_barrier()

    sem_g = (sem_g0, sem_g1)
    sem_s = (sem_s0, sem_s1)
    ones16 = jnp.ones((16,), jnp.float32)

    def run_block(ibs, ibd):
        hg = {}
        hs = {}
        hg[0] = pltpu.async_copy(
            h_hbm.at[ibs.at[0]], rows.at[pl.ds(0, CHUNK)], sem_g[0])
        for k in range(GR):
            slot = k % NSLOT
            if k + 1 < GR:
                nslot = (k + 1) % NSLOT
                if k - 1 >= 0:
                    hs[k - 1].wait()
                hg[k + 1] = pltpu.async_copy(
                    h_hbm.at[ibs.at[k + 1]],
                    rows.at[pl.ds(nslot * CHUNK, CHUNK)], sem_g[nslot])
            if with_deg:
                for c in range(CHUNK // 16):
                    v = ibd[k, pl.ds(c * 16, 16)]
                    row = lax.shift_right_logical(v, 7)
                    col = lax.bitwise_and(v, 127)
                    plsc.addupdate_scatter(hist, [row, col], ones16)
            hg[k].wait()
            hs[k] = pltpu.async_copy(
                rows.at[pl.ds(slot * CHUNK, CHUNK)],
                agg_sp.at[ibd.at[k]], sem_s[slot], add=True)
        hs[GR - 2].wait()
        hs[GR - 1].wait()

    pltpu.sync_copy(es_hbm.at[pl.ds(wbase, GR)], ibs0)
    pltpu.sync_copy(ed_hbm.at[pl.ds(wbase, GR)], ibd0)
    pltpu.async_copy(es_hbm.at[pl.ds(wbase + GR, GR)], ibs1, sem_i1)
    pltpu.async_copy(ed_hbm.at[pl.ds(wbase + GR, GR)], ibd1, sem_i1)

    @pl.loop(0, NBLK, step=2)
    def _(b):
        row0 = wbase + b * GR

        @pl.when(b > 0)
        def _():
            pltpu.make_async_copy(es_hbm.at[pl.ds(row0, GR)], ibs0,
                                  sem_i0).wait()
            pltpu.make_async_copy(ed_hbm.at[pl.ds(row0, GR)], ibd0,
                                  sem_i0).wait()

        run_block(ibs0, ibd0)
        pltpu.async_copy(es_hbm.at[pl.ds(row0 + 2 * GR, GR)], ibs0, sem_i0)
        pltpu.async_copy(ed_hbm.at[pl.ds(row0 + 2 * GR, GR)], ibd0, sem_i0)

        row1 = row0 + GR
        pltpu.make_async_copy(es_hbm.at[pl.ds(row1, GR)], ibs1, sem_i1).wait()
        pltpu.make_async_copy(ed_hbm.at[pl.ds(row1, GR)], ibd1, sem_i1).wait()
        run_block(ibs1, ibd1)
        pltpu.async_copy(es_hbm.at[pl.ds(row1 + 2 * GR, GR)], ibs1, sem_i1)
        pltpu.async_copy(ed_hbm.at[pl.ds(row1 + 2 * GR, GR)], ibd1, sem_i1)

    pltpu.make_async_copy(es_hbm.at[pl.ds(wbase, GR)], ibs0, sem_i0).wait()
    pltpu.make_async_copy(ed_hbm.at[pl.ds(wbase, GR)], ibd0, sem_i0).wait()
    pltpu.make_async_copy(es_hbm.at[pl.ds(wbase, GR)], ibs1, sem_i1).wait()
    pltpu.make_async_copy(ed_hbm.at[pl.ds(wbase, GR)], ibd1, sem_i1).wait()

    if with_deg:
        pltpu.sync_copy(hist, deg_sh.at[ident], add=True)

    plsc.subcore_barrier()
    pltpu.sync_copy(agg_sp.at[pl.ds(base, ZR)],
                    part_hbm.at[cidx].at[pl.ds(base, ZR)])
    if with_deg:
        @pl.when(sidx == 0)
        def _():
            pltpu.sync_copy(deg_sh, degd_hbm.at[cidx])


def _make_sc_agg(with_deg):
    out_type = [jax.ShapeDtypeStruct((NC, NSP, D), jnp.float32)]
    scratch = [
        pltpu.VMEM((GR, CHUNK), jnp.int32),
        pltpu.VMEM((GR, CHUNK), jnp.int32),
        pltpu.VMEM((GR, CHUNK), jnp.int32),
        pltpu.VMEM((GR, CHUNK), jnp.int32),
        pltpu.VMEM((NSLOT * CHUNK, D), jnp.float32),
    ]
    if with_deg:
        out_type.append(jax.ShapeDtypeStruct((NC, HR, D), jnp.float32))
        scratch.append(pltpu.VMEM((HR,), jnp.int32))
        scratch.append(pltpu.VMEM((HR, D), jnp.float32))
    scratch.append(pltpu.VMEM_SHARED((NSP, D), jnp.float32))
    if with_deg:
        scratch.append(pltpu.VMEM_SHARED((HR, D), jnp.float32))
    scratch += [pltpu.SemaphoreType.DMA] * 6
    return pl.kernel(
        functools.partial(_sc_agg_body, with_deg),
        out_type=tuple(out_type) if with_deg else out_type[0],
        mesh=_mesh,
        scratch_types=scratch,
        compiler_params=_cp,
    )


_sc_agg_deg = _make_sc_agg(True)
_sc_agg = _make_sc_agg(False)


RB = 1000
_PREC = lax.Precision.HIGHEST


def _tc_pre_body(h_ref, w_ref, b_ref, o_ref):
    o_ref[...] = lax.dot_general(
        h_ref[...], w_ref[...], (((1,), (0,)), ((), ())),
        precision=_PREC, preferred_element_type=jnp.float32) + b_ref[...]


def _tc_pre(h, w, b):
    return pl.pallas_call(
        _tc_pre_body,
        grid=(N // RB,),
        in_specs=[
            pl.BlockSpec((RB, D), lambda i: (i, 0)),
            pl.BlockSpec((D, D), lambda i: (0, 0)),
            pl.BlockSpec((1, D), lambda i: (0, 0)),
        ],
        out_specs=pl.BlockSpec((RB, D), lambda i: (i, 0)),
        out_shape=jax.ShapeDtypeStruct((N, D), jnp.float32),
    )(h, w[:D], b.reshape(1, D))


def _tc_post_body(relu, pre_ref, p_ref, d_ref, w_ref, o_ref):
    agg = p_ref[0] + p_ref[1]
    dsum = d_ref[0] + d_ref[1]
    deg = jnp.maximum(dsum, 1.0)
    mean = agg / deg
    y = pre_ref[...] + lax.dot_general(
        mean, w_ref[...], (((1,), (0,)), ((), ())),
        precision=_PREC, preferred_element_type=jnp.float32)
    if relu:
        y = jnp.maximum(y, 0.0)
    o_ref[...] = y


def _tc_post(pre, part, degc, w, relu):
    return pl.pallas_call(
        functools.partial(_tc_post_body, relu),
        grid=(N // RB,),
        in_specs=[
            pl.BlockSpec((RB, D), lambda i: (i, 0)),
            pl.BlockSpec((NC, RB, D), lambda i: (0, i, 0)),
            pl.BlockSpec((NC, RB, 1), lambda i: (0, i, 0)),
            pl.BlockSpec((D, D), lambda i: (0, 0)),
        ],
        out_specs=pl.BlockSpec((RB, D), lambda i: (i, 0)),
        out_shape=jax.ShapeDtypeStruct((N, D), jnp.float32),
    )(pre, part, degc, w[D:])


def kernel(x, edge_index, W0, b0, W1, b1):
    ei = edge_index.astype(jnp.int32)
    pr = RPF - ROWS
    r2 = (lax.broadcasted_iota(jnp.int32, (pr, CHUNK), 0) * CHUNK
          + lax.broadcasted_iota(jnp.int32, (pr, CHUNK), 1))
    pad_s = r2 % N
    pad_d = N + (r2 % (NSP - N))
    es = jnp.concatenate([ei[0].reshape(ROWS, CHUNK), pad_s], axis=0)
    ed = jnp.concatenate([ei[1].reshape(ROWS, CHUNK), pad_d], axis=0)
    ident = jnp.arange(HR, dtype=jnp.int32)

    part0, degd = _sc_agg_deg(x, es, ed, ident)
    degc = degd.reshape(NC, HR * D, 1)
    pre0 = _tc_pre(x, W0, b0)
    h1 = _tc_post(pre0, part0, degc, W0, relu=True)
    part1 = _sc_agg(h1, es, ed, ident)
    pre1 = _tc_pre(h1, W1, b1)
    h2 = _tc_post(pre1, part1, degc, W1, relu=False)
    return h2
